# baseline (speedup 1.0000x reference)
import functools

import jax
import jax.numpy as jnp
from jax import lax
from jax.experimental import pallas as pl
from jax.experimental.pallas import tpu as pltpu
from jax.experimental.pallas import tpu_sc as plsc

L = 196
LR = 200
D = 768
DC = 192
NK = 4
HW = 14


def _silu(v):
    return v * jax.nn.sigmoid(v)


def _softplus(v):
    return jnp.maximum(v, 0.0) + jnp.log1p(jnp.exp(-jnp.abs(v)))


def _ln(v, g, b, eps=1e-5):
    m = jnp.mean(v, axis=-1, keepdims=True)
    var = jnp.mean((v - m) * (v - m), axis=-1, keepdims=True)
    return (v - m) * lax.rsqrt(var + eps) * g + b


def _dot_t(a, b):
    return lax.dot_general(a, b, (((1,), (1,)), ((), ())),
                           preferred_element_type=jnp.float32)


def _pre_body(x_ref, si_ref, ipw_ref, w9_ref, cb_ref, xpw_ref, dtw_ref,
              dtb_ref, alog_ref, fea_ref, xc_ref, cc_ref, z_ref):
    nb = x_ref.shape[0]
    lidx = lax.broadcasted_iota(jnp.int32, (L, 1), 0)
    wmod = lidx % HW
    hdiv = lidx // HW
    liota = lax.broadcasted_iota(jnp.int32, (1, L), 1)
    w9 = w9_ref[...].T
    zpad = jnp.zeros((15, D), jnp.float32)
    zr = jnp.zeros((LR - L, D), jnp.float32)
    for bb in range(nb):
        xb = x_ref[bb].reshape(L, -1)
        xz = _dot_t(xb, ipw_ref[...])
        xp = xz[:, :D]
        z_ref[bb] = xz[:, D:].astype(jnp.bfloat16)

        xpad = jnp.concatenate([zpad, xp, zpad], axis=0)
        acc = jnp.zeros((L, D), jnp.float32)
        for di in (-1, 0, 1):
            for dj in (-1, 0, 1):
                s = di * HW + dj
                shifted = xpad[15 + s:15 + s + L, :]
                mask = None
                if dj == 1:
                    mask = wmod != (HW - 1)
                elif dj == -1:
                    mask = wmod != 0
                if di == 1:
                    mh = hdiv != (HW - 1)
                    mask = mh if mask is None else (mask & mh)
                elif di == -1:
                    mh = hdiv != 0
                    mask = mh if mask is None else (mask & mh)
                if mask is not None:
                    shifted = jnp.where(mask, shifted, 0.0)
                kidx = (di + 1) * 3 + (dj + 1)
                acc = acc + shifted * w9[kidx:kidx + 1, :]
        xc = _silu(acc + cb_ref[...])
        xc_ref[bb] = xc.astype(jnp.bfloat16)

        xdbl = _dot_t(xc, xpw_ref[...])
        bs = xdbl[:, 24:25]
        cc_ref[bb] = xdbl[:, 25:26]
        dts = _dot_t(xdbl[:, :24], dtw_ref[...])
        dts = _softplus(dts + dtb_ref[...])
        da = jnp.exp(dts * jnp.exp(alog_ref[...]))
        dbx = dts * bs * xc
        perm = (si_ref[bb] == liota).astype(jnp.float32)
        pfx = jnp.dot(perm, dbx, preferred_element_type=jnp.float32)
        pea = jnp.dot(perm, da, preferred_element_type=jnp.float32)
        pfx = jnp.concatenate([pfx, zr], axis=0)
        pea = jnp.concatenate([pea, zr], axis=0)
        for k in range(NK):
            cat = jnp.concatenate([pfx[:, k * DC:(k + 1) * DC],
                                   pea[:, k * DC:(k + 1) * DC]], axis=1)
            for tc in range(3):
                fea_ref[bb, k, tc] = cat[:, tc * 128:(tc + 1) * 128].reshape(
                    LR // 8, 8, 128)


def _sc_tree(fea_hbm, sp_hbm, out_hbm, fea_v, hb_v, sp_v):
    c = lax.axis_index("c")
    s = lax.axis_index("s")
    wid = s * 2 + c
    b = wid % 8
    k = wid // 8
    pltpu.sync_copy(fea_hbm.at[b, k], fea_v)
    pltpu.sync_copy(sp_hbm, sp_v)
    boff = b * L

    lanes = lax.iota(jnp.int32, 16)

    def step(t, carry):
        tv = jnp.full((16,), t, jnp.int32)
        bt = jnp.full((16,), boff + t, jnp.int32)
        par = plsc.load_gather(sp_v, [bt])
        valid = (par >= 0) & (par < tv)
        spar = jnp.maximum(par, 0)
        tr = t // 8
        sub = t % 8
        ptr = spar // 8
        psub = spar % 8
        for cc in range(DC // 16):
            tcf, lcf = divmod(cc * 16, 128)
            tce, lce = divmod(DC + cc * 16, 128)
            fx = fea_v[tcf, tr, sub, pl.ds(lcf, 16)]
            ea = fea_v[tce, tr, sub, pl.ds(lce, 16)]
            tch = jnp.full((16,), tcf, jnp.int32)
            hp = plsc.load_gather(hb_v, [tch, ptr, psub, lanes + lcf])
            hp = jnp.where(valid, hp, 0.0)
            hb_v[tcf, tr, sub, pl.ds(lcf, 16)] = ea * hp + fx
        return carry

    lax.fori_loop(0, L, step, 0, unroll=4)
    pltpu.sync_copy(hb_v, out_hbm.at[b, k])


def _post_body(h4_ref, sir_ref, cc_ref, xc_ref, z_ref, ds_ref, ong_ref,
               onb_ref, hng_ref, hnb_ref, opw_ref, out_ref):
    nb = h4_ref.shape[0]
    lcol = lax.broadcasted_iota(jnp.int32, (L, 1), 0)
    trow = lax.broadcasted_iota(jnp.int32, (1, L), 1)
    for bb in range(nb):
        pieces = []
        for k in range(NK):
            p0 = h4_ref[bb, k, 0].reshape(LR, 128)[:L]
            p1 = h4_ref[bb, k, 1].reshape(LR, 128)[:L, :DC - 128]
            pieces.append(p0)
            pieces.append(p1)
        hb = jnp.concatenate(pieces, axis=1)
        occ = sir_ref[bb] == lcol
        tl = jnp.max(jnp.where(occ, trow, -1), axis=1, keepdims=True)
        sel = (tl == trow).astype(jnp.float32)
        h = jnp.dot(sel, hb, preferred_element_type=jnp.float32)
        hn = _ln(h, hng_ref[...], hnb_ref[...])
        y = hn * cc_ref[bb] + ds_ref[...] * xc_ref[bb].astype(jnp.float32)
        y = _ln(y, ong_ref[...], onb_ref[...])
        y = y * _silu(z_ref[bb].astype(jnp.float32))
        out_ref[bb] = _dot_t(y, opw_ref[...])


@jax.jit
def kernel(x, bfs_indices, bfs_parents, in_proj_w, conv_w, conv_b,
           x_proj_weight, dt_projs_weight, dt_projs_bias, A_logs, Ds,
           out_norm_g, out_norm_b, h_norm_g, h_norm_b, out_proj_w):
    Bn, Hn, Wn, dm = x.shape
    w9 = conv_w.reshape(D, 9)
    cb = conv_b.reshape(1, D)
    xpw = x_proj_weight[0]
    dtw = dt_projs_weight[0]
    dtb = dt_projs_bias.reshape(1, D)
    alog = A_logs.reshape(1, D)
    ds2 = Ds.reshape(1, D)
    ong = out_norm_g.reshape(1, D)
    onb = out_norm_b.reshape(1, D)
    hng = h_norm_g.reshape(1, D)
    hnb = h_norm_b.reshape(1, D)
    si_col = bfs_indices.reshape(Bn, L, 1)
    si_row = bfs_indices.reshape(Bn, 1, L)
    sp_flat = bfs_parents.reshape(Bn * L)

    f32 = jnp.float32
    rep2 = lambda b: (0, 0)
    NBB = 4
    fea, xcv, ccv, zv = pl.pallas_call(
        _pre_body,
        grid=(Bn // NBB,),
        in_specs=[
            pl.BlockSpec((NBB, Hn, Wn, dm), lambda b: (b, 0, 0, 0)),
            pl.BlockSpec((NBB, L, 1), lambda b: (b, 0, 0)),
            pl.BlockSpec((2 * D, dm), rep2),
            pl.BlockSpec((D, 9), rep2),
            pl.BlockSpec((1, D), rep2),
            pl.BlockSpec((26, D), rep2),
            pl.BlockSpec((D, 24), rep2),
            pl.BlockSpec((1, D), rep2),
            pl.BlockSpec((1, D), rep2),
        ],
        out_specs=[
            pl.BlockSpec((NBB, NK, 3, LR // 8, 8, 128),
                         lambda b: (b, 0, 0, 0, 0, 0)),
            pl.BlockSpec((NBB, L, D), lambda b: (b, 0, 0)),
            pl.BlockSpec((NBB, L, 1), lambda b: (b, 0, 0)),
            pl.BlockSpec((NBB, L, D), lambda b: (b, 0, 0)),
        ],
        out_shape=[
            jax.ShapeDtypeStruct((Bn, NK, 3, LR // 8, 8, 128), f32),
            jax.ShapeDtypeStruct((Bn, L, D), jnp.bfloat16),
            jax.ShapeDtypeStruct((Bn, L, 1), f32),
            jax.ShapeDtypeStruct((Bn, L, D), jnp.bfloat16),
        ],
    )(x, si_col, in_proj_w, w9, cb, xpw, dtw, dtb, alog)

    sc_call = pl.kernel(
        _sc_tree,
        out_type=jax.ShapeDtypeStruct((Bn, NK, 2, LR // 8, 8, 128), f32),
        mesh=plsc.VectorSubcoreMesh(core_axis_name="c", subcore_axis_name="s",
                                    num_cores=2, num_subcores=16),
        compiler_params=pltpu.CompilerParams(needs_layout_passes=False,
                                             use_tc_tiling_on_sc=False),
        scratch_types=[
            pltpu.VMEM((3, LR // 8, 8, 128), f32),
            pltpu.VMEM((2, LR // 8, 8, 128), f32),
            pltpu.VMEM((Bn * L,), jnp.int32),
        ],
    )
    h4 = sc_call(fea, sp_flat)

    y = pl.pallas_call(
        _post_body,
        grid=(Bn // NBB,),
        in_specs=[
            pl.BlockSpec((NBB, NK, 2, LR // 8, 8, 128),
                         lambda b: (b, 0, 0, 0, 0, 0)),
            pl.BlockSpec((NBB, 1, L), lambda b: (b, 0, 0)),
            pl.BlockSpec((NBB, L, 1), lambda b: (b, 0, 0)),
            pl.BlockSpec((NBB, L, D), lambda b: (b, 0, 0)),
            pl.BlockSpec((NBB, L, D), lambda b: (b, 0, 0)),
            pl.BlockSpec((1, D), rep2),
            pl.BlockSpec((1, D), rep2),
            pl.BlockSpec((1, D), rep2),
            pl.BlockSpec((1, D), rep2),
            pl.BlockSpec((1, D), rep2),
            pl.BlockSpec((dm, D), rep2),
        ],
        out_specs=pl.BlockSpec((NBB, L, dm), lambda b: (b, 0, 0)),
        out_shape=jax.ShapeDtypeStruct((Bn, L, dm), f32),
    )(h4, si_row, ccv, xcv, zv, ds2, ong, onb, hng, hnb, out_proj_w)

    return y.reshape(Bn, Hn, Wn, dm)


# speedup vs baseline: 1.0035x; 1.0035x over previous
import functools

import jax
import jax.numpy as jnp
from jax import lax
from jax.experimental import pallas as pl
from jax.experimental.pallas import tpu as pltpu
from jax.experimental.pallas import tpu_sc as plsc

L = 196
LR = 200
D = 768
DC = 192
NK = 4
HW = 14


def _silu(v):
    return v * jax.nn.sigmoid(v)


def _softplus(v):
    return jnp.maximum(v, 0.0) + jnp.log1p(jnp.exp(-jnp.abs(v)))


def _ln(v, g, b, eps=1e-5):
    m = jnp.mean(v, axis=-1, keepdims=True)
    var = jnp.mean((v - m) * (v - m), axis=-1, keepdims=True)
    return (v - m) * lax.rsqrt(var + eps) * g + b


def _dot_t(a, b):
    return lax.dot_general(a, b, (((1,), (1,)), ((), ())),
                           preferred_element_type=jnp.float32)


def _pre_body(x_ref, si_ref, ipw_ref, w9_ref, cb_ref, xpw_ref, dtw_ref,
              dtb_ref, alog_ref, fea_ref, xc_ref, cc_ref, z_ref):
    nb = x_ref.shape[0]
    lidx = lax.broadcasted_iota(jnp.int32, (L, 1), 0)
    wmod = lidx % HW
    hdiv = lidx // HW
    liota = lax.broadcasted_iota(jnp.int32, (1, L), 1)
    w9 = w9_ref[...].T
    zpad = jnp.zeros((15, D), jnp.float32)
    zr = jnp.zeros((LR - L, D), jnp.float32)
    for bb in range(nb):
        xb = x_ref[bb].reshape(L, -1)
        xz = _dot_t(xb, ipw_ref[...])
        xp = xz[:, :D]
        z_ref[bb] = xz[:, D:].astype(jnp.bfloat16)

        xpad = jnp.concatenate([zpad, xp, zpad], axis=0)
        acc = jnp.zeros((L, D), jnp.float32)
        for di in (-1, 0, 1):
            for dj in (-1, 0, 1):
                s = di * HW + dj
                shifted = xpad[15 + s:15 + s + L, :]
                mask = None
                if dj == 1:
                    mask = wmod != (HW - 1)
                elif dj == -1:
                    mask = wmod != 0
                if di == 1:
                    mh = hdiv != (HW - 1)
                    mask = mh if mask is None else (mask & mh)
                elif di == -1:
                    mh = hdiv != 0
                    mask = mh if mask is None else (mask & mh)
                if mask is not None:
                    shifted = jnp.where(mask, shifted, 0.0)
                kidx = (di + 1) * 3 + (dj + 1)
                acc = acc + shifted * w9[kidx:kidx + 1, :]
        xc = _silu(acc + cb_ref[...])
        xc_ref[bb] = xc.astype(jnp.bfloat16)

        xdbl = _dot_t(xc, xpw_ref[...])
        bs = xdbl[:, 24:25]
        cc_ref[bb] = xdbl[:, 25:26]
        dts = _dot_t(xdbl[:, :24], dtw_ref[...])
        dts = _softplus(dts + dtb_ref[...])
        da = jnp.exp(dts * jnp.exp(alog_ref[...]))
        dbx = dts * bs * xc
        perm = (si_ref[bb] == liota).astype(jnp.float32)
        pfx = jnp.dot(perm, dbx, preferred_element_type=jnp.float32)
        pea = jnp.dot(perm, da, preferred_element_type=jnp.float32)
        pfx = jnp.concatenate([pfx, zr], axis=0)
        pea = jnp.concatenate([pea, zr], axis=0)
        for k in range(NK):
            cat = jnp.concatenate([pfx[:, k * DC:(k + 1) * DC],
                                   pea[:, k * DC:(k + 1) * DC]], axis=1)
            for tc in range(3):
                fea_ref[bb, k, tc] = cat[:, tc * 128:(tc + 1) * 128].reshape(
                    LR // 8, 8, 128)


def _sc_tree(fea_hbm, sp_hbm, out_hbm, fea_v, hb_v, sp_v):
    c = lax.axis_index("c")
    s = lax.axis_index("s")
    wid = s * 2 + c
    b = wid % 8
    k = wid // 8
    pltpu.sync_copy(fea_hbm.at[b, k], fea_v)
    pltpu.sync_copy(sp_hbm, sp_v)
    boff = b * L

    lanes = lax.iota(jnp.int32, 16)

    def step(t, carry):
        tv = jnp.full((16,), t, jnp.int32)
        bt = jnp.full((16,), boff + t, jnp.int32)
        par = plsc.load_gather(sp_v, [bt])
        valid = (par >= 0) & (par < tv)
        spar = jnp.maximum(par, 0)
        tr = t // 8
        sub = t % 8
        ptr = spar // 8
        psub = spar % 8
        for cc in range(DC // 16):
            tcf, lcf = divmod(cc * 16, 128)
            tce, lce = divmod(DC + cc * 16, 128)
            fx = fea_v[tcf, tr, sub, pl.ds(lcf, 16)]
            ea = fea_v[tce, tr, sub, pl.ds(lce, 16)]
            tch = jnp.full((16,), tcf, jnp.int32)
            hp = plsc.load_gather(hb_v, [tch, ptr, psub, lanes + lcf])
            hp = jnp.where(valid, hp, 0.0)
            hb_v[tcf, tr, sub, pl.ds(lcf, 16)] = ea * hp + fx
        return carry

    lax.fori_loop(0, L, step, 0, unroll=4)
    pltpu.sync_copy(hb_v, out_hbm.at[b, k])


def _post_body(h4_ref, sir_ref, cc_ref, xc_ref, z_ref, ds_ref, ong_ref,
               onb_ref, hng_ref, hnb_ref, opw_ref, out_ref):
    nb = h4_ref.shape[0]
    lcol = lax.broadcasted_iota(jnp.int32, (L, 1), 0)
    trow = lax.broadcasted_iota(jnp.int32, (1, L), 1)
    for bb in range(nb):
        pieces = []
        for k in range(NK):
            p0 = h4_ref[bb, k, 0].reshape(LR, 128)[:L]
            p1 = h4_ref[bb, k, 1].reshape(LR, 128)[:L, :DC - 128]
            pieces.append(p0)
            pieces.append(p1)
        hb = jnp.concatenate(pieces, axis=1)
        occ = sir_ref[bb] == lcol
        tl = jnp.max(jnp.where(occ, trow, -1), axis=1, keepdims=True)
        sel = (tl == trow).astype(jnp.float32)
        h = jnp.dot(sel, hb, preferred_element_type=jnp.float32)
        hn = _ln(h, hng_ref[...], hnb_ref[...])
        y = hn * cc_ref[bb] + ds_ref[...] * xc_ref[bb].astype(jnp.float32)
        y = _ln(y, ong_ref[...], onb_ref[...])
        y = y * _silu(z_ref[bb].astype(jnp.float32))
        out_ref[bb] = _dot_t(y, opw_ref[...])


@jax.jit
def kernel(x, bfs_indices, bfs_parents, in_proj_w, conv_w, conv_b,
           x_proj_weight, dt_projs_weight, dt_projs_bias, A_logs, Ds,
           out_norm_g, out_norm_b, h_norm_g, h_norm_b, out_proj_w):
    Bn, Hn, Wn, dm = x.shape
    w9 = conv_w.reshape(D, 9)
    cb = conv_b.reshape(1, D)
    xpw = x_proj_weight[0]
    dtw = dt_projs_weight[0]
    dtb = dt_projs_bias.reshape(1, D)
    alog = A_logs.reshape(1, D)
    ds2 = Ds.reshape(1, D)
    ong = out_norm_g.reshape(1, D)
    onb = out_norm_b.reshape(1, D)
    hng = h_norm_g.reshape(1, D)
    hnb = h_norm_b.reshape(1, D)
    si_col = bfs_indices.reshape(Bn, L, 1)
    si_row = bfs_indices.reshape(Bn, 1, L)
    sp_flat = bfs_parents.reshape(Bn * L)

    f32 = jnp.float32
    rep2 = lambda b: (0, 0)
    NBB = 2
    fea, xcv, ccv, zv = pl.pallas_call(
        _pre_body,
        grid=(Bn // NBB,),
        in_specs=[
            pl.BlockSpec((NBB, Hn, Wn, dm), lambda b: (b, 0, 0, 0)),
            pl.BlockSpec((NBB, L, 1), lambda b: (b, 0, 0)),
            pl.BlockSpec((2 * D, dm), rep2),
            pl.BlockSpec((D, 9), rep2),
            pl.BlockSpec((1, D), rep2),
            pl.BlockSpec((26, D), rep2),
            pl.BlockSpec((D, 24), rep2),
            pl.BlockSpec((1, D), rep2),
            pl.BlockSpec((1, D), rep2),
        ],
        out_specs=[
            pl.BlockSpec((NBB, NK, 3, LR // 8, 8, 128),
                         lambda b: (b, 0, 0, 0, 0, 0)),
            pl.BlockSpec((NBB, L, D), lambda b: (b, 0, 0)),
            pl.BlockSpec((NBB, L, 1), lambda b: (b, 0, 0)),
            pl.BlockSpec((NBB, L, D), lambda b: (b, 0, 0)),
        ],
        out_shape=[
            jax.ShapeDtypeStruct((Bn, NK, 3, LR // 8, 8, 128), f32),
            jax.ShapeDtypeStruct((Bn, L, D), jnp.bfloat16),
            jax.ShapeDtypeStruct((Bn, L, 1), f32),
            jax.ShapeDtypeStruct((Bn, L, D), jnp.bfloat16),
        ],
    )(x, si_col, in_proj_w, w9, cb, xpw, dtw, dtb, alog)

    sc_call = pl.kernel(
        _sc_tree,
        out_type=jax.ShapeDtypeStruct((Bn, NK, 2, LR // 8, 8, 128), f32),
        mesh=plsc.VectorSubcoreMesh(core_axis_name="c", subcore_axis_name="s",
                                    num_cores=2, num_subcores=16),
        compiler_params=pltpu.CompilerParams(needs_layout_passes=False,
                                             use_tc_tiling_on_sc=False),
        scratch_types=[
            pltpu.VMEM((3, LR // 8, 8, 128), f32),
            pltpu.VMEM((2, LR // 8, 8, 128), f32),
            pltpu.VMEM((Bn * L,), jnp.int32),
        ],
    )
    h4 = sc_call(fea, sp_flat)

    y = pl.pallas_call(
        _post_body,
        grid=(Bn // NBB,),
        in_specs=[
            pl.BlockSpec((NBB, NK, 2, LR // 8, 8, 128),
                         lambda b: (b, 0, 0, 0, 0, 0)),
            pl.BlockSpec((NBB, 1, L), lambda b: (b, 0, 0)),
            pl.BlockSpec((NBB, L, 1), lambda b: (b, 0, 0)),
            pl.BlockSpec((NBB, L, D), lambda b: (b, 0, 0)),
            pl.BlockSpec((NBB, L, D), lambda b: (b, 0, 0)),
            pl.BlockSpec((1, D), rep2),
            pl.BlockSpec((1, D), rep2),
            pl.BlockSpec((1, D), rep2),
            pl.BlockSpec((1, D), rep2),
            pl.BlockSpec((1, D), rep2),
            pl.BlockSpec((dm, D), rep2),
        ],
        out_specs=pl.BlockSpec((NBB, L, dm), lambda b: (b, 0, 0)),
        out_shape=jax.ShapeDtypeStruct((Bn, L, dm), f32),
    )(h4, si_row, ccv, xcv, zv, ds2, ong, onb, hng, hnb, out_proj_w)

    return y.reshape(Bn, Hn, Wn, dm)


# speedup vs baseline: 1.0392x; 1.0356x over previous
import functools

import jax
import jax.numpy as jnp
from jax import lax
from jax.experimental import pallas as pl
from jax.experimental.pallas import tpu as pltpu
from jax.experimental.pallas import tpu_sc as plsc

L = 196
LR = 200
D = 768
DC = 192
NK = 4
HW = 14


def _silu(v):
    return v * jax.nn.sigmoid(v)


def _softplus(v):
    return jnp.maximum(v, 0.0) + jnp.log1p(jnp.exp(-jnp.abs(v)))


def _ln(v, g, b, eps=1e-5):
    m = jnp.mean(v, axis=-1, keepdims=True)
    var = jnp.mean((v - m) * (v - m), axis=-1, keepdims=True)
    return (v - m) * lax.rsqrt(var + eps) * g + b


def _dot_t(a, b):
    return lax.dot_general(a, b, (((1,), (1,)), ((), ())),
                           preferred_element_type=jnp.float32)


def _pre_body(x_ref, si_ref, ipw_ref, w9_ref, cb_ref, xpw_ref, dtw_ref,
              dtb_ref, alog_ref, fea_ref, xc_ref, cc_ref, z_ref):
    nb = x_ref.shape[0]
    lidx = lax.broadcasted_iota(jnp.int32, (L, 1), 0)
    wmod = lidx % HW
    hdiv = lidx // HW
    liota = lax.broadcasted_iota(jnp.int32, (1, L), 1)
    w9 = w9_ref[...].T
    zpad = jnp.zeros((15, D), jnp.float32)
    zr = jnp.zeros((LR - L, D), jnp.float32)
    for bb in range(nb):
        xb = x_ref[bb].reshape(L, -1)
        xz = _dot_t(xb, ipw_ref[...])
        xp = xz[:, :D]
        z_ref[bb] = xz[:, D:].astype(jnp.bfloat16)

        xpad = jnp.concatenate([zpad, xp, zpad], axis=0)
        acc = jnp.zeros((L, D), jnp.float32)
        for di in (-1, 0, 1):
            for dj in (-1, 0, 1):
                s = di * HW + dj
                shifted = xpad[15 + s:15 + s + L, :]
                mask = None
                if dj == 1:
                    mask = wmod != (HW - 1)
                elif dj == -1:
                    mask = wmod != 0
                if di == 1:
                    mh = hdiv != (HW - 1)
                    mask = mh if mask is None else (mask & mh)
                elif di == -1:
                    mh = hdiv != 0
                    mask = mh if mask is None else (mask & mh)
                if mask is not None:
                    shifted = jnp.where(mask, shifted, 0.0)
                kidx = (di + 1) * 3 + (dj + 1)
                acc = acc + shifted * w9[kidx:kidx + 1, :]
        xc = _silu(acc + cb_ref[...])
        xc_ref[bb] = xc.astype(jnp.bfloat16)

        xdbl = _dot_t(xc, xpw_ref[...])
        bs = xdbl[:, 24:25]
        cc_ref[bb] = xdbl[:, 25:26]
        dts = _dot_t(xdbl[:, :24], dtw_ref[...])
        dts = _softplus(dts + dtb_ref[...])
        da = jnp.exp(dts * jnp.exp(alog_ref[...]))
        dbx = dts * bs * xc
        perm = (si_ref[bb] == liota).astype(jnp.float32)
        pfx = jnp.dot(perm, dbx, preferred_element_type=jnp.float32)
        pea = jnp.dot(perm, da, preferred_element_type=jnp.float32)
        pfx = jnp.concatenate([pfx, zr], axis=0)
        pea = jnp.concatenate([pea, zr], axis=0)
        for k in range(NK):
            cat = jnp.concatenate([pfx[:, k * DC:(k + 1) * DC],
                                   pea[:, k * DC:(k + 1) * DC]], axis=1)
            for tc in range(3):
                fea_ref[bb, k, tc] = cat[:, tc * 128:(tc + 1) * 128].reshape(
                    LR // 8, 8, 128)


def _sc_tree(fea_hbm, sp_hbm, out_hbm, fea_v, hb_v, sp_v):
    c = lax.axis_index("c")
    s = lax.axis_index("s")
    wid = s * 2 + c
    b = wid % 8
    k = wid // 8
    pltpu.sync_copy(fea_hbm.at[b, k], fea_v)
    pltpu.sync_copy(sp_hbm, sp_v)
    boff = b * L

    lanes = lax.iota(jnp.int32, 16)

    def step(t, par):
        tv = jnp.full((16,), t, jnp.int32)
        valid = (par >= 0) & (par < tv)
        spar = jnp.maximum(par, 0)
        tr = t // 8
        sub = t % 8
        ptr = spar // 8
        psub = spar % 8
        bt1 = jnp.full((16,), jnp.minimum(boff + t + 1, sp_v.shape[0] - 1),
                       jnp.int32)
        par_next = plsc.load_gather(sp_v, [bt1])
        hps = []
        for cc in range(DC // 16):
            tcf, lcf = divmod(cc * 16, 128)
            tch = jnp.full((16,), tcf, jnp.int32)
            hp = plsc.load_gather(hb_v, [tch, ptr, psub, lanes + lcf])
            hps.append(jnp.where(valid, hp, 0.0))
        for cc in range(DC // 16):
            tcf, lcf = divmod(cc * 16, 128)
            tce, lce = divmod(DC + cc * 16, 128)
            fx = fea_v[tcf, tr, sub, pl.ds(lcf, 16)]
            ea = fea_v[tce, tr, sub, pl.ds(lce, 16)]
            hb_v[tcf, tr, sub, pl.ds(lcf, 16)] = ea * hps[cc] + fx
        return par_next

    par0 = plsc.load_gather(sp_v, [jnp.full((16,), boff, jnp.int32)])
    lax.fori_loop(0, L, step, par0, unroll=4)
    pltpu.sync_copy(hb_v, out_hbm.at[b, k])


def _post_body(h4_ref, sir_ref, cc_ref, xc_ref, z_ref, ds_ref, ong_ref,
               onb_ref, hng_ref, hnb_ref, opw_ref, out_ref):
    nb = h4_ref.shape[0]
    lcol = lax.broadcasted_iota(jnp.int32, (L, 1), 0)
    trow = lax.broadcasted_iota(jnp.int32, (1, L), 1)
    for bb in range(nb):
        pieces = []
        for k in range(NK):
            p0 = h4_ref[bb, k, 0].reshape(LR, 128)[:L]
            p1 = h4_ref[bb, k, 1].reshape(LR, 128)[:L, :DC - 128]
            pieces.append(p0)
            pieces.append(p1)
        hb = jnp.concatenate(pieces, axis=1)
        occ = sir_ref[bb] == lcol
        tl = jnp.max(jnp.where(occ, trow, -1), axis=1, keepdims=True)
        sel = (tl == trow).astype(jnp.float32)
        h = jnp.dot(sel, hb, preferred_element_type=jnp.float32)
        hn = _ln(h, hng_ref[...], hnb_ref[...])
        y = hn * cc_ref[bb] + ds_ref[...] * xc_ref[bb].astype(jnp.float32)
        y = _ln(y, ong_ref[...], onb_ref[...])
        y = y * _silu(z_ref[bb].astype(jnp.float32))
        out_ref[bb] = _dot_t(y, opw_ref[...])


@jax.jit
def kernel(x, bfs_indices, bfs_parents, in_proj_w, conv_w, conv_b,
           x_proj_weight, dt_projs_weight, dt_projs_bias, A_logs, Ds,
           out_norm_g, out_norm_b, h_norm_g, h_norm_b, out_proj_w):
    Bn, Hn, Wn, dm = x.shape
    w9 = conv_w.reshape(D, 9)
    cb = conv_b.reshape(1, D)
    xpw = x_proj_weight[0]
    dtw = dt_projs_weight[0]
    dtb = dt_projs_bias.reshape(1, D)
    alog = A_logs.reshape(1, D)
    ds2 = Ds.reshape(1, D)
    ong = out_norm_g.reshape(1, D)
    onb = out_norm_b.reshape(1, D)
    hng = h_norm_g.reshape(1, D)
    hnb = h_norm_b.reshape(1, D)
    si_col = bfs_indices.reshape(Bn, L, 1)
    si_row = bfs_indices.reshape(Bn, 1, L)
    sp_flat = bfs_parents.reshape(Bn * L)

    f32 = jnp.float32
    rep2 = lambda b: (0, 0)
    NBB = 2
    fea, xcv, ccv, zv = pl.pallas_call(
        _pre_body,
        grid=(Bn // NBB,),
        in_specs=[
            pl.BlockSpec((NBB, Hn, Wn, dm), lambda b: (b, 0, 0, 0)),
            pl.BlockSpec((NBB, L, 1), lambda b: (b, 0, 0)),
            pl.BlockSpec((2 * D, dm), rep2),
            pl.BlockSpec((D, 9), rep2),
            pl.BlockSpec((1, D), rep2),
            pl.BlockSpec((26, D), rep2),
            pl.BlockSpec((D, 24), rep2),
            pl.BlockSpec((1, D), rep2),
            pl.BlockSpec((1, D), rep2),
        ],
        out_specs=[
            pl.BlockSpec((NBB, NK, 3, LR // 8, 8, 128),
                         lambda b: (b, 0, 0, 0, 0, 0)),
            pl.BlockSpec((NBB, L, D), lambda b: (b, 0, 0)),
            pl.BlockSpec((NBB, L, 1), lambda b: (b, 0, 0)),
            pl.BlockSpec((NBB, L, D), lambda b: (b, 0, 0)),
        ],
        out_shape=[
            jax.ShapeDtypeStruct((Bn, NK, 3, LR // 8, 8, 128), f32),
            jax.ShapeDtypeStruct((Bn, L, D), jnp.bfloat16),
            jax.ShapeDtypeStruct((Bn, L, 1), f32),
            jax.ShapeDtypeStruct((Bn, L, D), jnp.bfloat16),
        ],
    )(x, si_col, in_proj_w, w9, cb, xpw, dtw, dtb, alog)

    sc_call = pl.kernel(
        _sc_tree,
        out_type=jax.ShapeDtypeStruct((Bn, NK, 2, LR // 8, 8, 128), f32),
        mesh=plsc.VectorSubcoreMesh(core_axis_name="c", subcore_axis_name="s",
                                    num_cores=2, num_subcores=16),
        compiler_params=pltpu.CompilerParams(needs_layout_passes=False,
                                             use_tc_tiling_on_sc=False),
        scratch_types=[
            pltpu.VMEM((3, LR // 8, 8, 128), f32),
            pltpu.VMEM((2, LR // 8, 8, 128), f32),
            pltpu.VMEM((Bn * L,), jnp.int32),
        ],
    )
    h4 = sc_call(fea, sp_flat)

    y = pl.pallas_call(
        _post_body,
        grid=(Bn // NBB,),
        in_specs=[
            pl.BlockSpec((NBB, NK, 2, LR // 8, 8, 128),
                         lambda b: (b, 0, 0, 0, 0, 0)),
            pl.BlockSpec((NBB, 1, L), lambda b: (b, 0, 0)),
            pl.BlockSpec((NBB, L, 1), lambda b: (b, 0, 0)),
            pl.BlockSpec((NBB, L, D), lambda b: (b, 0, 0)),
            pl.BlockSpec((NBB, L, D), lambda b: (b, 0, 0)),
            pl.BlockSpec((1, D), rep2),
            pl.BlockSpec((1, D), rep2),
            pl.BlockSpec((1, D), rep2),
            pl.BlockSpec((1, D), rep2),
            pl.BlockSpec((1, D), rep2),
            pl.BlockSpec((dm, D), rep2),
        ],
        out_specs=pl.BlockSpec((NBB, L, dm), lambda b: (b, 0, 0)),
        out_shape=jax.ShapeDtypeStruct((Bn, L, dm), f32),
    )(h4, si_row, ccv, xcv, zv, ds2, ong, onb, hng, hnb, out_proj_w)

    return y.reshape(Bn, Hn, Wn, dm)


# speedup vs baseline: 1.0604x; 1.0205x over previous
import functools

import jax
import jax.numpy as jnp
from jax import lax
from jax.experimental import pallas as pl
from jax.experimental.pallas import tpu as pltpu
from jax.experimental.pallas import tpu_sc as plsc

L = 196
LR = 200
D = 768
DC = 192
NK = 4
HW = 14


def _silu(v):
    return v * jax.nn.sigmoid(v)


def _softplus(v):
    return jnp.maximum(v, 0.0) + jnp.log1p(jnp.exp(-jnp.abs(v)))


def _ln(v, g, b, eps=1e-5):
    m = jnp.mean(v, axis=-1, keepdims=True)
    var = jnp.mean((v - m) * (v - m), axis=-1, keepdims=True)
    return (v - m) * lax.rsqrt(var + eps) * g + b


def _dot_t(a, b):
    return lax.dot_general(a, b, (((1,), (1,)), ((), ())),
                           preferred_element_type=jnp.float32)


def _pre_body(x_ref, si_ref, ipw_ref, w9_ref, cb_ref, xpw_ref, dtw_ref,
              dtb_ref, alog_ref, fea_ref, xc_ref, cc_ref, z_ref):
    nb = x_ref.shape[0]
    lidx = lax.broadcasted_iota(jnp.int32, (L, 1), 0)
    wmod = lidx % HW
    hdiv = lidx // HW
    liota = lax.broadcasted_iota(jnp.int32, (1, L), 1)
    w9 = w9_ref[...].T
    zpad = jnp.zeros((15, D), jnp.float32)
    zr = jnp.zeros((LR - L, 2 * D), jnp.float32)
    for bb in range(nb):
        xb = x_ref[bb].reshape(L, -1)
        xz = _dot_t(xb, ipw_ref[...])
        xp = xz[:, :D]
        z_ref[bb] = xz[:, D:].astype(jnp.bfloat16)

        xpad = jnp.concatenate([zpad, xp, zpad], axis=0)
        xmsk = {0: xpad}
        pwm = jnp.concatenate([jnp.zeros((15, 1), jnp.int32), wmod,
                               jnp.zeros((15, 1), jnp.int32)], axis=0)
        xmsk[1] = jnp.where(pwm != 0, xpad, 0.0)
        xmsk[-1] = jnp.where(pwm != (HW - 1), xpad, 0.0)
        acc = jnp.zeros((L, D), jnp.float32)
        for di in (-1, 0, 1):
            for dj in (-1, 0, 1):
                s = di * HW + dj
                shifted = xmsk[dj][15 + s:15 + s + L, :]
                kidx = (di + 1) * 3 + (dj + 1)
                acc = acc + shifted * w9[kidx:kidx + 1, :]
        xc = _silu(acc + cb_ref[...])
        xc_ref[bb] = xc.astype(jnp.bfloat16)

        xdbl = _dot_t(xc, xpw_ref[...])
        bs = xdbl[:, 24:25]
        cc_ref[bb] = xdbl[:, 25:26]
        dts = _dot_t(xdbl[:, :24], dtw_ref[...])
        dts = _softplus(dts + dtb_ref[...])
        da = jnp.exp(dts * jnp.exp(alog_ref[...]))
        dbx = dts * bs * xc
        perm = (si_ref[bb] == liota).astype(jnp.float32)
        both = jnp.concatenate([dbx, da], axis=1)
        pboth = jnp.dot(perm, both, preferred_element_type=jnp.float32)
        pboth = jnp.concatenate([pboth, zr], axis=0)
        for k in range(NK):
            cat = jnp.concatenate([pboth[:, k * DC:(k + 1) * DC],
                                   pboth[:, D + k * DC:D + (k + 1) * DC]],
                                  axis=1)
            for tc in range(3):
                fea_ref[bb, k, tc] = cat[:, tc * 128:(tc + 1) * 128].reshape(
                    LR // 8, 8, 128)


def _sc_tree(fea_hbm, sp_hbm, out_hbm, fea_v, hb_v, sp_v):
    c = lax.axis_index("c")
    s = lax.axis_index("s")
    wid = s * 2 + c
    b = wid % 8
    k = wid // 8
    pltpu.sync_copy(fea_hbm.at[b, k], fea_v)
    pltpu.sync_copy(sp_hbm, sp_v)
    boff = b * L

    lanes = lax.iota(jnp.int32, 16)

    def step(t, par):
        tv = jnp.full((16,), t, jnp.int32)
        valid = (par >= 0) & (par < tv)
        spar = jnp.maximum(par, 0)
        tr = t // 8
        sub = t % 8
        ptr = spar // 8
        psub = spar % 8
        bt1 = jnp.full((16,), jnp.minimum(boff + t + 1, sp_v.shape[0] - 1),
                       jnp.int32)
        par_next = plsc.load_gather(sp_v, [bt1])
        hps = []
        for cc in range(DC // 16):
            tcf, lcf = divmod(cc * 16, 128)
            tch = jnp.full((16,), tcf, jnp.int32)
            hp = plsc.load_gather(hb_v, [tch, ptr, psub, lanes + lcf])
            hps.append(jnp.where(valid, hp, 0.0))
        for cc in range(DC // 16):
            tcf, lcf = divmod(cc * 16, 128)
            tce, lce = divmod(DC + cc * 16, 128)
            fx = fea_v[tcf, tr, sub, pl.ds(lcf, 16)]
            ea = fea_v[tce, tr, sub, pl.ds(lce, 16)]
            hb_v[tcf, tr, sub, pl.ds(lcf, 16)] = ea * hps[cc] + fx
        return par_next

    par0 = plsc.load_gather(sp_v, [jnp.full((16,), boff, jnp.int32)])
    lax.fori_loop(0, L, step, par0, unroll=4)
    pltpu.sync_copy(hb_v, out_hbm.at[b, k])


def _post_body(h4_ref, sir_ref, cc_ref, xc_ref, z_ref, ds_ref, ong_ref,
               onb_ref, hng_ref, hnb_ref, opw_ref, out_ref):
    nb = h4_ref.shape[0]
    lcol = lax.broadcasted_iota(jnp.int32, (L, 1), 0)
    trow = lax.broadcasted_iota(jnp.int32, (1, L), 1)
    for bb in range(nb):
        pieces = []
        for k in range(NK):
            p0 = h4_ref[bb, k, 0].reshape(LR, 128)[:L]
            p1 = h4_ref[bb, k, 1].reshape(LR, 128)[:L, :DC - 128]
            pieces.append(p0)
            pieces.append(p1)
        hb = jnp.concatenate(pieces, axis=1)
        occ = sir_ref[bb] == lcol
        tl = jnp.max(jnp.where(occ, trow, -1), axis=1, keepdims=True)
        sel = (tl == trow).astype(jnp.float32)
        h = jnp.dot(sel, hb, preferred_element_type=jnp.float32)
        hn = _ln(h, hng_ref[...], hnb_ref[...])
        y = hn * cc_ref[bb] + ds_ref[...] * xc_ref[bb].astype(jnp.float32)
        y = _ln(y, ong_ref[...], onb_ref[...])
        y = y * _silu(z_ref[bb].astype(jnp.float32))
        out_ref[bb] = _dot_t(y, opw_ref[...])


@jax.jit
def kernel(x, bfs_indices, bfs_parents, in_proj_w, conv_w, conv_b,
           x_proj_weight, dt_projs_weight, dt_projs_bias, A_logs, Ds,
           out_norm_g, out_norm_b, h_norm_g, h_norm_b, out_proj_w):
    Bn, Hn, Wn, dm = x.shape
    w9 = conv_w.reshape(D, 9)
    cb = conv_b.reshape(1, D)
    xpw = x_proj_weight[0]
    dtw = dt_projs_weight[0]
    dtb = dt_projs_bias.reshape(1, D)
    alog = A_logs.reshape(1, D)
    ds2 = Ds.reshape(1, D)
    ong = out_norm_g.reshape(1, D)
    onb = out_norm_b.reshape(1, D)
    hng = h_norm_g.reshape(1, D)
    hnb = h_norm_b.reshape(1, D)
    si_col = bfs_indices.reshape(Bn, L, 1)
    si_row = bfs_indices.reshape(Bn, 1, L)
    sp_flat = bfs_parents.reshape(Bn * L)

    f32 = jnp.float32
    rep2 = lambda b: (0, 0)
    NBB = 2
    fea, xcv, ccv, zv = pl.pallas_call(
        _pre_body,
        grid=(Bn // NBB,),
        in_specs=[
            pl.BlockSpec((NBB, Hn, Wn, dm), lambda b: (b, 0, 0, 0)),
            pl.BlockSpec((NBB, L, 1), lambda b: (b, 0, 0)),
            pl.BlockSpec((2 * D, dm), rep2),
            pl.BlockSpec((D, 9), rep2),
            pl.BlockSpec((1, D), rep2),
            pl.BlockSpec((26, D), rep2),
            pl.BlockSpec((D, 24), rep2),
            pl.BlockSpec((1, D), rep2),
            pl.BlockSpec((1, D), rep2),
        ],
        out_specs=[
            pl.BlockSpec((NBB, NK, 3, LR // 8, 8, 128),
                         lambda b: (b, 0, 0, 0, 0, 0)),
            pl.BlockSpec((NBB, L, D), lambda b: (b, 0, 0)),
            pl.BlockSpec((NBB, L, 1), lambda b: (b, 0, 0)),
            pl.BlockSpec((NBB, L, D), lambda b: (b, 0, 0)),
        ],
        out_shape=[
            jax.ShapeDtypeStruct((Bn, NK, 3, LR // 8, 8, 128), f32),
            jax.ShapeDtypeStruct((Bn, L, D), jnp.bfloat16),
            jax.ShapeDtypeStruct((Bn, L, 1), f32),
            jax.ShapeDtypeStruct((Bn, L, D), jnp.bfloat16),
        ],
    )(x, si_col, in_proj_w, w9, cb, xpw, dtw, dtb, alog)

    sc_call = pl.kernel(
        _sc_tree,
        out_type=jax.ShapeDtypeStruct((Bn, NK, 2, LR // 8, 8, 128), f32),
        mesh=plsc.VectorSubcoreMesh(core_axis_name="c", subcore_axis_name="s",
                                    num_cores=2, num_subcores=16),
        compiler_params=pltpu.CompilerParams(needs_layout_passes=False,
                                             use_tc_tiling_on_sc=False),
        scratch_types=[
            pltpu.VMEM((3, LR // 8, 8, 128), f32),
            pltpu.VMEM((2, LR // 8, 8, 128), f32),
            pltpu.VMEM((Bn * L,), jnp.int32),
        ],
    )
    h4 = sc_call(fea, sp_flat)

    y = pl.pallas_call(
        _post_body,
        grid=(Bn // NBB,),
        in_specs=[
            pl.BlockSpec((NBB, NK, 2, LR // 8, 8, 128),
                         lambda b: (b, 0, 0, 0, 0, 0)),
            pl.BlockSpec((NBB, 1, L), lambda b: (b, 0, 0)),
            pl.BlockSpec((NBB, L, 1), lambda b: (b, 0, 0)),
            pl.BlockSpec((NBB, L, D), lambda b: (b, 0, 0)),
            pl.BlockSpec((NBB, L, D), lambda b: (b, 0, 0)),
            pl.BlockSpec((1, D), rep2),
            pl.BlockSpec((1, D), rep2),
            pl.BlockSpec((1, D), rep2),
            pl.BlockSpec((1, D), rep2),
            pl.BlockSpec((1, D), rep2),
            pl.BlockSpec((dm, D), rep2),
        ],
        out_specs=pl.BlockSpec((NBB, L, dm), lambda b: (b, 0, 0)),
        out_shape=jax.ShapeDtypeStruct((Bn, L, dm), f32),
    )(h4, si_row, ccv, xcv, zv, ds2, ong, onb, hng, hnb, out_proj_w)

    return y.reshape(Bn, Hn, Wn, dm)
